# baseline (speedup 1.0000x reference)
import jax
import jax.numpy as jnp
from jax.experimental import pallas as pl
from jax.experimental.pallas import tpu as pltpu

N = 1024
NFEAT = 128
NHID = 16
NOUT = 128
NHEADS = 8
ALPHA = 0.2
NBLK = 8
BLK = N // NBLK


def _masked_weights(adj_blk, b_row, d_row, p_col):
    r = jnp.exp((1.0 - ALPHA) * p_col)
    return adj_blk * jnp.minimum(b_row, r * d_row)


def _elu(v):
    return jnp.where(v > 0, v, jnp.exp(v) - 1.0)


def _gat_kernel(x_ref, adj_ref, wall_ref, a1_ref, a2_ref, wout_ref, ao_ref,
                out_ref, hall_s, x2_s, h2_s, b1_s, d1_s, b2_s, d2_s):
    i = pl.program_id(0)
    ones_col = jnp.ones((N, 1), dtype=jnp.float32)

    @pl.when(i == 0)
    def _prep1():
        h_all = jnp.dot(x_ref[...], wall_ref[...],
                        preferred_element_type=jnp.float32)
        hall_s[...] = h_all
        for hd in range(NHEADS):
            h_i = h_all[:, hd * NHID:(hd + 1) * NHID]
            q = jax.lax.dot_general(a2_ref[hd:hd + 1, :], h_i,
                                    (((1,), (1,)), ((), ())),
                                    preferred_element_type=jnp.float32)
            b1_s[hd:hd + 1, :] = jnp.exp(-q)
            d1_s[hd:hd + 1, :] = jnp.exp(-ALPHA * q)

    @pl.when(i < NBLK)
    def _layer1():
        adj_blk = adj_ref[...]
        h_blk = hall_s[pl.ds(i * BLK, BLK), :]
        outs = []
        for hd in range(NHEADS):
            h_i_blk = h_blk[:, hd * NHID:(hd + 1) * NHID]
            p = jax.lax.dot_general(h_i_blk, a1_ref[hd:hd + 1, :],
                                    (((1,), (1,)), ((), ())),
                                    preferred_element_type=jnp.float32)
            e = _masked_weights(adj_blk, b1_s[hd:hd + 1, :],
                                d1_s[hd:hd + 1, :], p)
            h_aug = jnp.concatenate(
                [hall_s[:, hd * NHID:(hd + 1) * NHID], ones_col], axis=1)
            nd = jnp.dot(e, h_aug, preferred_element_type=jnp.float32)
            hp = nd[:, :NHID] * (1.0 / nd[:, NHID:NHID + 1])
            outs.append(_elu(hp))
        x2_s[pl.ds(i * BLK, BLK), :] = jnp.concatenate(outs, axis=1)

    @pl.when(i == NBLK)
    def _prep2():
        h2 = jnp.dot(x2_s[...], wout_ref[...],
                     preferred_element_type=jnp.float32)
        h2_s[...] = h2
        q2 = jax.lax.dot_general(ao_ref[:, NOUT:], h2,
                                 (((1,), (1,)), ((), ())),
                                 preferred_element_type=jnp.float32)
        b2_s[...] = jnp.exp(-q2)
        d2_s[...] = jnp.exp(-ALPHA * q2)

    @pl.when(i >= NBLK)
    def _layer2():
        j = i - NBLK
        adj_blk = adj_ref[...]
        h2_blk = h2_s[pl.ds(j * BLK, BLK), :]
        p2 = jax.lax.dot_general(h2_blk, ao_ref[:, :NOUT],
                                 (((1,), (1,)), ((), ())),
                                 preferred_element_type=jnp.float32)
        e2 = _masked_weights(adj_blk, b2_s[...], d2_s[...], p2)
        h2_aug = jnp.concatenate([h2_s[...], ones_col], axis=1)
        nd = jnp.dot(e2, h2_aug, preferred_element_type=jnp.float32)
        h_out = nd[:, :NOUT] * (1.0 / nd[:, NOUT:NOUT + 1])
        deg = jnp.sum(adj_blk, axis=1, keepdims=True)
        x_blk = x_ref[pl.ds(j * BLK, BLK), :]
        out_ref[...] = _elu(jnp.where(deg == 0.0, x_blk, h_out))


def kernel(x, adj, W_heads, a_heads, W_out, a_out):
    w_all = jnp.transpose(W_heads, (1, 0, 2)).reshape(NFEAT, NHEADS * NHID)
    a1_all = a_heads[:, 0, :NHID]
    a2_all = a_heads[:, 0, NHID:]
    f32 = jnp.float32
    return pl.pallas_call(
        _gat_kernel,
        grid=(2 * NBLK,),
        in_specs=[
            pl.BlockSpec((N, NFEAT), lambda i: (0, 0)),
            pl.BlockSpec((BLK, N), lambda i: (i % NBLK, 0)),
            pl.BlockSpec((NFEAT, NHEADS * NHID), lambda i: (0, 0)),
            pl.BlockSpec((NHEADS, NHID), lambda i: (0, 0)),
            pl.BlockSpec((NHEADS, NHID), lambda i: (0, 0)),
            pl.BlockSpec((NHEADS * NHID, NOUT), lambda i: (0, 0)),
            pl.BlockSpec((1, 2 * NOUT), lambda i: (0, 0)),
        ],
        out_specs=pl.BlockSpec((BLK, NOUT), lambda i: (i % NBLK, 0)),
        out_shape=jax.ShapeDtypeStruct((N, NOUT), f32),
        scratch_shapes=[
            pltpu.VMEM((N, NHEADS * NHID), f32),
            pltpu.VMEM((N, NHEADS * NHID), f32),
            pltpu.VMEM((N, NOUT), f32),
            pltpu.VMEM((NHEADS, N), f32),
            pltpu.VMEM((NHEADS, N), f32),
            pltpu.VMEM((1, N), f32),
            pltpu.VMEM((1, N), f32),
        ],
    )(x, adj, w_all, a1_all, a2_all, W_out, a_out)


# speedup vs baseline: 1.4079x; 1.4079x over previous
import jax
import jax.numpy as jnp
from jax.experimental import pallas as pl

N = 1024
NFEAT = 128
NHID = 16
NOUT = 128
NHEADS = 8
ALPHA = 0.2


def _elu(v):
    return jnp.where(v > 0, v, jnp.exp(v) - 1.0)


def _agg(h, p, q, adj_bf, ones_bf):
    b = jnp.exp(-q).astype(jnp.bfloat16)
    db = jnp.exp(-ALPHA * q).astype(jnp.bfloat16)
    r = jnp.exp((1.0 - ALPHA) * p).astype(jnp.bfloat16)
    e = adj_bf * jnp.minimum(b, r * db)
    h_aug = jnp.concatenate([h, ones_bf], axis=1).astype(jnp.bfloat16)
    nd = jnp.dot(e, h_aug, preferred_element_type=jnp.float32)
    d = h.shape[1]
    return nd[:, :d] * (1.0 / nd[:, d:d + 1])


def _gat_kernel(x_ref, adj_ref, wall_ref, a1_ref, a2_ref, wout_ref, ao_ref,
                out_ref):
    x = x_ref[...]
    adj_bf = adj_ref[...]
    ones_bf = jnp.ones((N, 1), dtype=jnp.float32)

    h_all = jnp.dot(x, wall_ref[...], preferred_element_type=jnp.float32)
    head_outs = []
    for i in range(NHEADS):
        h_i = h_all[:, i * NHID:(i + 1) * NHID]
        a1 = a1_ref[i:i + 1, :]
        a2 = a2_ref[i:i + 1, :]
        p = jax.lax.dot_general(h_i, a1, (((1,), (1,)), ((), ())),
                                preferred_element_type=jnp.float32)
        q = jax.lax.dot_general(a2, h_i, (((1,), (1,)), ((), ())),
                                preferred_element_type=jnp.float32)
        head_outs.append(_elu(_agg(h_i, p, q, adj_bf, ones_bf)))
    x2 = jnp.concatenate(head_outs, axis=1)

    h2 = jnp.dot(x2, wout_ref[...], preferred_element_type=jnp.float32)
    a1o = ao_ref[:, :NOUT]
    a2o = ao_ref[:, NOUT:]
    p2 = jax.lax.dot_general(h2, a1o, (((1,), (1,)), ((), ())),
                             preferred_element_type=jnp.float32)
    q2 = jax.lax.dot_general(a2o, h2, (((1,), (1,)), ((), ())),
                             preferred_element_type=jnp.float32)
    h_out = _agg(h2, p2, q2, adj_bf, ones_bf)

    deg = jnp.dot(adj_bf, ones_bf.astype(jnp.bfloat16),
                  preferred_element_type=jnp.float32)
    h_out = jnp.where(deg == 0.0, x, h_out)
    out_ref[...] = _elu(h_out)


def kernel(x, adj, W_heads, a_heads, W_out, a_out):
    w_all = jnp.transpose(W_heads, (1, 0, 2)).reshape(NFEAT, NHEADS * NHID)
    a1_all = a_heads[:, 0, :NHID]
    a2_all = a_heads[:, 0, NHID:]
    adj_bf = adj.astype(jnp.bfloat16)
    return pl.pallas_call(
        _gat_kernel,
        out_shape=jax.ShapeDtypeStruct((N, NOUT), jnp.float32),
    )(x, adj_bf, w_all, a1_all, a2_all, W_out, a_out)
